# Initial kernel scaffold; baseline (speedup 1.0000x reference)
import jax
import jax.numpy as jnp
from jax.experimental import pallas as pl


def kernel(x, node_attrs, edge_radial, edge_spherical, edge_index, W_lin, W_fc1, W_fc2, W_sc):
    raise NotImplementedError("write your pallas kernel here")


# speedup vs baseline: 1.9900x; 1.9900x over previous
import functools

import jax
import jax.numpy as jnp
from jax import lax
from jax.experimental import pallas as pl
from jax.experimental.pallas import tpu as pltpu
from jax.experimental.pallas import tpu_sc as plsc

N = 10000
E = 320000
C = 128
A = 16
R = 8
H = 8
AVG_NEIGH = 32.0

NC = 2
NS = 16
L = 16

NB = 1000
EB = 4000
K = 80
EPW = E // (NC * NS)
NBLK = EPW // K
RPT = 624
ZROWS = 208
TAIL = N - NS * RPT


def _node_kernel(x_ref, na_ref, wl_ref, wsc_ref, xl_ref, sc_ref):
    xb = x_ref[...]
    xl_ref[...] = jnp.dot(xb, wl_ref[...] * (1.0 / jnp.sqrt(float(C))),
                          preferred_element_type=jnp.float32)
    acc = jnp.zeros((NB, C), dtype=jnp.float32)
    for a in range(A):
        acc = acc + jnp.dot(xb * na_ref[:, a][:, None], wsc_ref[:, a, :],
                            preferred_element_type=jnp.float32)
    sc_ref[...] = acc * (1.0 / jnp.sqrt(float(C * A)))


def _edge_kernel(rad_ref, sph_ref, wf1_ref, wf2_ref, w2_ref):
    pre = jnp.dot(rad_ref[...], wf1_ref[...] * (1.0 / jnp.sqrt(float(R))),
                  preferred_element_type=jnp.float32)
    h = jax.nn.softplus(pre) - jnp.log(2.0)
    w = jnp.dot(h, wf2_ref[...] * (1.0 / jnp.sqrt(float(H))),
                preferred_element_type=jnp.float32)
    w2_ref[...] = w * sph_ref[...] * (1.0 / jnp.sqrt(AVG_NEIGH))


def _combine_kernel(a0_ref, a1_ref, sc_ref, out_ref):
    out_ref[...] = a0_ref[...] + a1_ref[...] + sc_ref[...]


def _sc_kernel(xl_hbm, w2_hbm, src_hbm, dst_hbm, out_hbm,
               acc, srcb, dstb, rows, w2b, zb, sem):
    cid = lax.axis_index("c")
    sid = lax.axis_index("s")

    def zrow(i, _):
        for k in range(C // L):
            zb[i, pl.ds(k * L, L)] = jnp.zeros((L,), jnp.float32)
        return 0
    lax.fori_loop(0, ZROWS, zrow, 0)
    r0 = sid * RPT
    for j in range(RPT // ZROWS):
        pltpu.sync_copy(zb, acc.at[pl.ds(r0 + j * ZROWS, ZROWS)])

    @pl.when(sid == NS - 1)
    def _():
        pltpu.sync_copy(zb.at[pl.ds(0, TAIL)], acc.at[pl.ds(NS * RPT, TAIL)])
    plsc.subcore_barrier()

    base_e = (cid * NS + sid) * EPW

    def body(b, _):
        e0 = base_e + b * K
        pltpu.sync_copy(src_hbm.at[pl.ds(e0, K)], srcb)
        pltpu.sync_copy(dst_hbm.at[pl.ds(e0, K)], dstb)
        pltpu.async_copy(xl_hbm.at[srcb], rows, sem).wait()
        pltpu.sync_copy(w2_hbm.at[pl.ds(e0, K)], w2b)

        def mul(i, _):
            for k in range(C // L):
                sl = pl.ds(k * L, L)
                rows[i, sl] = rows[i, sl] * w2b[i, sl]
            return 0
        lax.fori_loop(0, K, mul, 0)

        pltpu.sync_copy(rows, acc.at[dstb], add=True)
        return 0
    lax.fori_loop(0, NBLK, body, 0)
    plsc.subcore_barrier()

    pltpu.sync_copy(acc.at[pl.ds(r0, RPT)], out_hbm.at[cid, pl.ds(r0, RPT)])

    @pl.when(sid == NS - 1)
    def _():
        pltpu.sync_copy(acc.at[pl.ds(NS * RPT, TAIL)],
                        out_hbm.at[cid, pl.ds(NS * RPT, TAIL)])


def kernel(x, node_attrs, edge_radial, edge_spherical, edge_index,
           W_lin, W_fc1, W_fc2, W_sc):
    xl, sc = pl.pallas_call(
        _node_kernel,
        grid=(N // NB,),
        in_specs=[
            pl.BlockSpec((NB, C), lambda i: (i, 0)),
            pl.BlockSpec((NB, A), lambda i: (i, 0)),
            pl.BlockSpec((C, C), lambda i: (0, 0)),
            pl.BlockSpec((C, A, C), lambda i: (0, 0, 0)),
        ],
        out_specs=[
            pl.BlockSpec((NB, C), lambda i: (i, 0)),
            pl.BlockSpec((NB, C), lambda i: (i, 0)),
        ],
        out_shape=[
            jax.ShapeDtypeStruct((N, C), jnp.float32),
            jax.ShapeDtypeStruct((N, C), jnp.float32),
        ],
    )(x, node_attrs, W_lin, W_sc)

    w2 = pl.pallas_call(
        _edge_kernel,
        grid=(E // EB,),
        in_specs=[
            pl.BlockSpec((EB, R), lambda i: (i, 0)),
            pl.BlockSpec((EB, 1), lambda i: (i, 0)),
            pl.BlockSpec((R, H), lambda i: (0, 0)),
            pl.BlockSpec((H, C), lambda i: (0, 0)),
        ],
        out_specs=pl.BlockSpec((EB, C), lambda i: (i, 0)),
        out_shape=jax.ShapeDtypeStruct((E, C), jnp.float32),
    )(edge_radial, edge_spherical, W_fc1, W_fc2)

    src = edge_index[0]
    dst = edge_index[1]
    acc = functools.partial(
        pl.kernel,
        out_type=jax.ShapeDtypeStruct((NC, N, C), jnp.float32),
        mesh=plsc.VectorSubcoreMesh(core_axis_name="c", subcore_axis_name="s",
                                    num_cores=NC, num_subcores=NS),
        scratch_types=[
            pltpu.VMEM_SHARED((N, C), jnp.float32),
            pltpu.VMEM((K,), jnp.int32),
            pltpu.VMEM((K,), jnp.int32),
            pltpu.VMEM((K, C), jnp.float32),
            pltpu.VMEM((K, C), jnp.float32),
            pltpu.VMEM((ZROWS, C), jnp.float32),
            pltpu.SemaphoreType.DMA,
        ],
    )(_sc_kernel)(xl, w2, src, dst)

    out = pl.pallas_call(
        _combine_kernel,
        grid=(N // NB,),
        in_specs=[
            pl.BlockSpec((NB, C), lambda i: (i, 0)),
            pl.BlockSpec((NB, C), lambda i: (i, 0)),
            pl.BlockSpec((NB, C), lambda i: (i, 0)),
        ],
        out_specs=pl.BlockSpec((NB, C), lambda i: (i, 0)),
        out_shape=jax.ShapeDtypeStruct((N, C), jnp.float32),
    )(acc[0], acc[1], sc)
    return out


# speedup vs baseline: 3.0157x; 1.5154x over previous
import functools

import jax
import jax.numpy as jnp
from jax import lax
from jax.experimental import pallas as pl
from jax.experimental.pallas import tpu as pltpu
from jax.experimental.pallas import tpu_sc as plsc

N = 10000
E = 320000
C = 128
A = 16
R = 8
H = 8
AVG_NEIGH = 32.0

NC = 2
NS = 16
L = 16

NB = 1000
EB = 4000
K = 80
EPW = E // (NC * NS)
NBLK = EPW // K
RPT = 624
ZROWS = 208
TAIL = N - NS * RPT


def _node_kernel(x_ref, na_ref, wl_ref, wsc_ref, xl_ref, sc_ref):
    xb = x_ref[...]
    xl_ref[...] = jnp.dot(xb, wl_ref[...] * (1.0 / jnp.sqrt(float(C))),
                          preferred_element_type=jnp.float32)
    acc = jnp.zeros((NB, C), dtype=jnp.float32)
    for a in range(A):
        acc = acc + jnp.dot(xb * na_ref[:, a][:, None], wsc_ref[:, a, :],
                            preferred_element_type=jnp.float32)
    sc_ref[...] = acc * (1.0 / jnp.sqrt(float(C * A)))


def _edge_kernel(rad_ref, sph_ref, wf1_ref, wf2_ref, w2_ref):
    pre = jnp.dot(rad_ref[...], wf1_ref[...] * (1.0 / jnp.sqrt(float(R))),
                  preferred_element_type=jnp.float32)
    h = jax.nn.softplus(pre) - jnp.log(2.0)
    w = jnp.dot(h, wf2_ref[...] * (1.0 / jnp.sqrt(float(H))),
                preferred_element_type=jnp.float32)
    w2_ref[...] = w * sph_ref[...] * (1.0 / jnp.sqrt(AVG_NEIGH))


def _combine_kernel(a0_ref, a1_ref, sc_ref, out_ref):
    out_ref[...] = a0_ref[...] + a1_ref[...] + sc_ref[...]


def _sc_kernel(xl_hbm, w2_hbm, src_hbm, dst_hbm, out_hbm,
               acc, srcb0, srcb1, srcb2, dstb0, dstb1, dstb2, rows0, rows1,
               w2b0, w2b1, sg0, sg1, sw0, sw1, si0, si1, si2, sd0, sd1, sd2):
    cid = lax.axis_index("c")
    sid = lax.axis_index("s")
    rows = (rows0, rows1)
    w2b = (w2b0, w2b1)
    srcb = (srcb0, srcb1, srcb2)
    dstb = (dstb0, dstb1, dstb2)
    sg = (sg0, sg1)
    sw = (sw0, sw1)
    si = (si0, si1, si2)
    sd = (sd0, sd1, sd2)

    def zrow(i, _):
        for k in range(C // L):
            rows0[i, pl.ds(k * L, L)] = jnp.zeros((L,), jnp.float32)
        return 0
    lax.fori_loop(0, K, zrow, 0)
    r0 = sid * RPT
    for j in range(RPT // K):
        pltpu.sync_copy(rows0, acc.at[pl.ds(r0 + j * K, K)])
    rem = RPT - (RPT // K) * K
    if rem:
        pltpu.sync_copy(rows0.at[pl.ds(0, rem)],
                        acc.at[pl.ds(r0 + (RPT // K) * K, rem)])

    @pl.when(sid == NS - 1)
    def _():
        pltpu.sync_copy(rows0.at[pl.ds(0, TAIL)], acc.at[pl.ds(NS * RPT, TAIL)])
    plsc.subcore_barrier()

    base_e = (cid * NS + sid) * EPW

    def issue_idx(t, b):
        pltpu.async_copy(src_hbm.at[pl.ds(base_e + b * K, K)], srcb[t], si[t])
        pltpu.async_copy(dst_hbm.at[pl.ds(base_e + b * K, K)], dstb[t], sd[t])

    def wait_idx(t, b):
        pltpu.make_async_copy(src_hbm.at[pl.ds(base_e + b * K, K)],
                              srcb[t], si[t]).wait()
        pltpu.make_async_copy(dst_hbm.at[pl.ds(base_e + b * K, K)],
                              dstb[t], sd[t]).wait()

    def issue_gw(s, t, b):
        pltpu.async_copy(w2_hbm.at[pl.ds(base_e + b * K, K)], w2b[s], sw[s])
        pltpu.async_copy(xl_hbm.at[srcb[t]], rows[s], sg[s])

    def step(j, b, nxt_gw, nxt_idx):
        s, t = j % 2, j % 3
        if nxt_idx:
            issue_idx((j + 2) % 3, b + 2)
        pltpu.make_async_copy(w2_hbm.at[pl.ds(base_e + b * K, K)],
                              w2b[s], sw[s]).wait()
        pltpu.make_async_copy(xl_hbm.at[srcb[t]], rows[s], sg[s]).wait()
        if nxt_gw:
            wait_idx((j + 1) % 3, b + 1)
            issue_gw((j + 1) % 2, (j + 1) % 3, b + 1)

        def mul(i, _):
            for k in range(C // L):
                sl = pl.ds(k * L, L)
                rows[s][i, sl] = rows[s][i, sl] * w2b[s][i, sl]
            return 0
        lax.fori_loop(0, K, mul, 0)
        pltpu.sync_copy(rows[s], acc.at[dstb[t]], add=True)

    issue_idx(0, 0)
    issue_idx(1, 1)
    wait_idx(0, 0)
    issue_gw(0, 0, 0)

    NMAIN = (NBLK - 5) // 6 * 6

    def body(i, _):
        b0 = 6 * i
        for j in range(6):
            step(j, b0 + j, True, True)
        return 0
    lax.fori_loop(0, NMAIN // 6, body, 0)
    for b in range(NMAIN, NBLK):
        step(b % 6, b, b + 1 < NBLK, b + 2 < NBLK)
    plsc.subcore_barrier()

    pltpu.sync_copy(acc.at[pl.ds(r0, RPT)], out_hbm.at[cid, pl.ds(r0, RPT)])

    @pl.when(sid == NS - 1)
    def _():
        pltpu.sync_copy(acc.at[pl.ds(NS * RPT, TAIL)],
                        out_hbm.at[cid, pl.ds(NS * RPT, TAIL)])


def kernel(x, node_attrs, edge_radial, edge_spherical, edge_index,
           W_lin, W_fc1, W_fc2, W_sc):
    xl, sc = pl.pallas_call(
        _node_kernel,
        grid=(N // NB,),
        in_specs=[
            pl.BlockSpec((NB, C), lambda i: (i, 0)),
            pl.BlockSpec((NB, A), lambda i: (i, 0)),
            pl.BlockSpec((C, C), lambda i: (0, 0)),
            pl.BlockSpec((C, A, C), lambda i: (0, 0, 0)),
        ],
        out_specs=[
            pl.BlockSpec((NB, C), lambda i: (i, 0)),
            pl.BlockSpec((NB, C), lambda i: (i, 0)),
        ],
        out_shape=[
            jax.ShapeDtypeStruct((N, C), jnp.float32),
            jax.ShapeDtypeStruct((N, C), jnp.float32),
        ],
    )(x, node_attrs, W_lin, W_sc)

    w2 = pl.pallas_call(
        _edge_kernel,
        grid=(E // EB,),
        in_specs=[
            pl.BlockSpec((EB, R), lambda i: (i, 0)),
            pl.BlockSpec((EB, 1), lambda i: (i, 0)),
            pl.BlockSpec((R, H), lambda i: (0, 0)),
            pl.BlockSpec((H, C), lambda i: (0, 0)),
        ],
        out_specs=pl.BlockSpec((EB, C), lambda i: (i, 0)),
        out_shape=jax.ShapeDtypeStruct((E, C), jnp.float32),
    )(edge_radial, edge_spherical, W_fc1, W_fc2)

    src = edge_index[0]
    dst = edge_index[1]
    acc = functools.partial(
        pl.kernel,
        out_type=jax.ShapeDtypeStruct((NC, N, C), jnp.float32),
        mesh=plsc.VectorSubcoreMesh(core_axis_name="c", subcore_axis_name="s",
                                    num_cores=NC, num_subcores=NS),
        scratch_types=(
            [pltpu.VMEM_SHARED((N, C), jnp.float32)]
            + [pltpu.VMEM((K,), jnp.int32)] * 6
            + [pltpu.VMEM((K, C), jnp.float32)] * 4
            + [pltpu.SemaphoreType.DMA] * 10
        ),
    )(_sc_kernel)(xl, w2, src, dst)

    out = pl.pallas_call(
        _combine_kernel,
        grid=(N // NB,),
        in_specs=[
            pl.BlockSpec((NB, C), lambda i: (i, 0)),
            pl.BlockSpec((NB, C), lambda i: (i, 0)),
            pl.BlockSpec((NB, C), lambda i: (i, 0)),
        ],
        out_specs=pl.BlockSpec((NB, C), lambda i: (i, 0)),
        out_shape=jax.ShapeDtypeStruct((N, C), jnp.float32),
    )(acc[0], acc[1], sc)
    return out


# speedup vs baseline: 4.5331x; 1.5032x over previous
import functools

import jax
import jax.numpy as jnp
from jax import lax
from jax.experimental import pallas as pl
from jax.experimental.pallas import tpu as pltpu
from jax.experimental.pallas import tpu_sc as plsc

N = 10000
E = 320000
C = 128
A = 16
R = 8
H = 8
AVG_NEIGH = 32.0

NC = 2
NS = 16
L = 16

NB = 1000
EB = 6400
K = 80
EPW = E // (NC * NS)
NBLK = EPW // K
RPT = 624
ZROWS = 208
TAIL = N - NS * RPT


def _node_kernel(x_ref, na_ref, wl_ref, wsc_ref, xl_ref, sc_ref):
    xb = x_ref[...]
    xl_ref[...] = jnp.dot(xb, wl_ref[...] * (1.0 / jnp.sqrt(float(C))),
                          preferred_element_type=jnp.float32)
    acc = jnp.zeros((NB, C), dtype=jnp.float32)
    for a in range(A):
        acc = acc + jnp.dot(xb * na_ref[:, a][:, None], wsc_ref[:, a, :],
                            preferred_element_type=jnp.float32)
    sc_ref[...] = acc * (1.0 / jnp.sqrt(float(C * A)))


def _edge_kernel(radT_ref, wf1_ref, wf2_ref, w2_ref):
    pre = lax.dot_general(radT_ref[...],
                          wf1_ref[...] * (1.0 / jnp.sqrt(float(R))),
                          (((0,), (0,)), ((), ())),
                          preferred_element_type=jnp.float32)
    h = jax.nn.softplus(pre) - jnp.log(2.0)
    w2_ref[...] = jnp.dot(
        h, wf2_ref[...] * (1.0 / jnp.sqrt(float(H) * AVG_NEIGH)),
        preferred_element_type=jnp.float32)


def _combine_kernel(acc_ref, sc_ref, out_ref):
    out_ref[...] = acc_ref[0] + acc_ref[1] + sc_ref[...]


def _sc_kernel(xl_hbm, w2_hbm, sph_hbm, src_hbm, dst_hbm, out_hbm,
               acc, srcb0, srcb1, srcb2, dstb0, dstb1, dstb2, rows0, rows1,
               w2b0, w2b1, sphb0, sphb1,
               sg0, sg1, sw0, sw1, si0, si1, si2, sd0, sd1, sd2, sp0, sp1):
    cid = lax.axis_index("c")
    sid = lax.axis_index("s")
    rows = (rows0, rows1)
    w2b = (w2b0, w2b1)
    sphb = (sphb0, sphb1)
    srcb = (srcb0, srcb1, srcb2)
    dstb = (dstb0, dstb1, dstb2)
    sg = (sg0, sg1)
    sw = (sw0, sw1)
    si = (si0, si1, si2)
    sd = (sd0, sd1, sd2)
    sp = (sp0, sp1)

    def zrow(i, _):
        for k in range(C // L):
            rows0[i, pl.ds(k * L, L)] = jnp.zeros((L,), jnp.float32)
        return 0
    lax.fori_loop(0, K, zrow, 0)
    r0 = sid * RPT
    for j in range(RPT // K):
        pltpu.sync_copy(rows0, acc.at[pl.ds(r0 + j * K, K)])
    rem = RPT - (RPT // K) * K
    if rem:
        pltpu.sync_copy(rows0.at[pl.ds(0, rem)],
                        acc.at[pl.ds(r0 + (RPT // K) * K, rem)])

    @pl.when(sid == NS - 1)
    def _():
        pltpu.sync_copy(rows0.at[pl.ds(0, TAIL)], acc.at[pl.ds(NS * RPT, TAIL)])
    plsc.subcore_barrier()

    base_e = (cid * NS + sid) * EPW

    def issue_idx(t, b):
        pltpu.async_copy(src_hbm.at[pl.ds(base_e + b * K, K)], srcb[t], si[t])
        pltpu.async_copy(dst_hbm.at[pl.ds(base_e + b * K, K)], dstb[t], sd[t])

    def wait_idx(t, b):
        pltpu.make_async_copy(src_hbm.at[pl.ds(base_e + b * K, K)],
                              srcb[t], si[t]).wait()
        pltpu.make_async_copy(dst_hbm.at[pl.ds(base_e + b * K, K)],
                              dstb[t], sd[t]).wait()

    def issue_gw(s, t, b):
        pltpu.async_copy(w2_hbm.at[pl.ds(base_e + b * K, K)], w2b[s], sw[s])
        pltpu.async_copy(sph_hbm.at[pl.ds(base_e + b * K, K)], sphb[s].at[pl.ds(0, K)], sp[s])
        pltpu.async_copy(xl_hbm.at[srcb[t]], rows[s], sg[s])

    def step(j, b, nxt_gw, nxt_idx):
        s, t = j % 2, j % 3
        if nxt_idx:
            issue_idx((j + 2) % 3, b + 2)
        pltpu.make_async_copy(w2_hbm.at[pl.ds(base_e + b * K, K)],
                              w2b[s], sw[s]).wait()
        pltpu.make_async_copy(sph_hbm.at[pl.ds(base_e + b * K, K)],
                              sphb[s].at[pl.ds(0, K)], sp[s]).wait()
        pltpu.make_async_copy(xl_hbm.at[srcb[t]], rows[s], sg[s]).wait()
        if nxt_gw:
            wait_idx((j + 1) % 3, b + 1)
            issue_gw((j + 1) % 2, (j + 1) % 3, b + 1)

        def mul(i, _):
            sv = sphb[s][pl.ds(i, L)][0]
            for k in range(C // L):
                sl = pl.ds(k * L, L)
                rows[s][i, sl] = rows[s][i, sl] * w2b[s][i, sl] * sv
            return 0
        lax.fori_loop(0, K, mul, 0)
        pltpu.sync_copy(rows[s], acc.at[dstb[t]], add=True)

    issue_idx(0, 0)
    issue_idx(1, 1)
    wait_idx(0, 0)
    issue_gw(0, 0, 0)

    NMAIN = (NBLK - 5) // 6 * 6

    def body(i, _):
        b0 = 6 * i
        for j in range(6):
            step(j, b0 + j, True, True)
        return 0
    lax.fori_loop(0, NMAIN // 6, body, 0)
    for b in range(NMAIN, NBLK):
        step(b % 6, b, b + 1 < NBLK, b + 2 < NBLK)
    plsc.subcore_barrier()

    pltpu.sync_copy(acc.at[pl.ds(r0, RPT)], out_hbm.at[cid, pl.ds(r0, RPT)])

    @pl.when(sid == NS - 1)
    def _():
        pltpu.sync_copy(acc.at[pl.ds(NS * RPT, TAIL)],
                        out_hbm.at[cid, pl.ds(NS * RPT, TAIL)])


def kernel(x, node_attrs, edge_radial, edge_spherical, edge_index,
           W_lin, W_fc1, W_fc2, W_sc):
    xl, sc = pl.pallas_call(
        _node_kernel,
        grid=(N // NB,),
        in_specs=[
            pl.BlockSpec((NB, C), lambda i: (i, 0)),
            pl.BlockSpec((NB, A), lambda i: (i, 0)),
            pl.BlockSpec((C, C), lambda i: (0, 0)),
            pl.BlockSpec((C, A, C), lambda i: (0, 0, 0)),
        ],
        out_specs=[
            pl.BlockSpec((NB, C), lambda i: (i, 0)),
            pl.BlockSpec((NB, C), lambda i: (i, 0)),
        ],
        out_shape=[
            jax.ShapeDtypeStruct((N, C), jnp.float32),
            jax.ShapeDtypeStruct((N, C), jnp.float32),
        ],
    )(x, node_attrs, W_lin, W_sc)

    w2 = pl.pallas_call(
        _edge_kernel,
        grid=(E // EB,),
        in_specs=[
            pl.BlockSpec((R, EB), lambda i: (0, i)),
            pl.BlockSpec((R, H), lambda i: (0, 0)),
            pl.BlockSpec((H, C), lambda i: (0, 0)),
        ],
        out_specs=pl.BlockSpec((EB, C), lambda i: (i, 0)),
        out_shape=jax.ShapeDtypeStruct((E, C), jnp.float32),
    )(edge_radial.T, W_fc1, W_fc2)

    src = edge_index[0]
    dst = edge_index[1]
    sph = edge_spherical.reshape(E)
    acc = functools.partial(
        pl.kernel,
        out_type=jax.ShapeDtypeStruct((NC, N, C), jnp.float32),
        mesh=plsc.VectorSubcoreMesh(core_axis_name="c", subcore_axis_name="s",
                                    num_cores=NC, num_subcores=NS),
        scratch_types=(
            [pltpu.VMEM_SHARED((N, C), jnp.float32)]
            + [pltpu.VMEM((K,), jnp.int32)] * 6
            + [pltpu.VMEM((K, C), jnp.float32)] * 4
            + [pltpu.VMEM((K + L,), jnp.float32)] * 2
            + [pltpu.SemaphoreType.DMA] * 12
        ),
    )(_sc_kernel)(xl, w2, sph, src, dst)

    out = pl.pallas_call(
        _combine_kernel,
        grid=(N // NB,),
        in_specs=[
            pl.BlockSpec((NC, NB, C), lambda i: (0, i, 0)),
            pl.BlockSpec((NB, C), lambda i: (i, 0)),
        ],
        out_specs=pl.BlockSpec((NB, C), lambda i: (i, 0)),
        out_shape=jax.ShapeDtypeStruct((N, C), jnp.float32),
    )(acc, sc)
    return out
